# baseline (speedup 1.0000x reference)
import jax
import jax.numpy as jnp
from jax import lax
from jax.experimental import pallas as pl
from jax.experimental.pallas import tpu as pltpu
from jax.experimental.pallas import tpu_sc as plsc

NC = 2
NS = 16
L = 16
NW = NC * NS
B = 16384
K = 32
BPW = B // NW
W = 128
NSLOT = 17
CHUNK = 128
NCHUNK = BPW // CHUNK


def _extract_body(su_hbm, si_hbm, nfu_hbm, nfi_hbm, wdu_hbm, wdi_hbm,
                  ut_hbm, it_hbm, uemb_hbm, iemb_hbm,
                  idx_v, nf_v, wd_v, slabs, ebuf, sem):
    cid = lax.axis_index("c")
    sid = lax.axis_index("s")
    wid = sid * NC + cid
    base = wid * BPW
    iota = lax.iota(jnp.int32, L)

    def one_pass(sv_hbm, nf_hbm, wdx_hbm, tab, out_hbm):
        pltpu.sync_copy(sv_hbm.at[pl.ds(base, BPW)], idx_v)
        pltpu.sync_copy(nf_hbm.at[pl.ds(base, BPW)], nf_v)
        pltpu.sync_copy(wdx_hbm.at[pl.ds(base, BPW)], wd_v)

        def block(g, carry):
            svv = idx_v[pl.ds(g * L, L)]
            nfv = nf_v[pl.ds(g * L, L)]
            wdv = wd_v[pl.ds(g * L, L)]
            for i in range(L):
                r = svv[i]
                slot = lax.rem(wdv[i], NSLOT)
                cb = pl.multiple_of(r - lax.rem(r, W), W)

                @pl.when(nfv[i] == 1)
                def _fire():
                    pltpu.async_copy(tab.at[:, pl.ds(cb, W)],
                                     slabs.at[slot], sem)
            for i in range(L):
                @pl.when(nfv[i] == 1)
                def _drain():
                    pltpu.make_async_copy(tab.at[:, pl.ds(0, W)],
                                          slabs.at[0], sem).wait()
            for i in range(L):
                r = svv[i]
                slot = lax.rem(wdv[i], NSLOT)
                j = lax.rem(r, W)
                sl = jnp.full((L,), slot, jnp.int32)
                jj = jnp.full((L,), j, jnp.int32)
                lo = plsc.load_gather(slabs, [sl, iota, jj])
                hi = plsc.load_gather(slabs, [sl, iota + L, jj])
                q = g * L + i
                ebuf[pl.ds(q * K, L)] = lo
                ebuf[pl.ds(q * K + L, L)] = hi
            return carry

        lax.fori_loop(0, BPW // L, block, 0)
        pltpu.sync_copy(ebuf, out_hbm.at[pl.ds(base * K, BPW * K)])

    one_pass(su_hbm, nfu_hbm, wdu_hbm, ut_hbm, uemb_hbm)
    one_pass(si_hbm, nfi_hbm, wdi_hbm, it_hbm, iemb_hbm)


def _pair_body(upos_hbm, ipos_hbm, ue_hbm, ie_hbm, out_hbm,
               uidx_v, iidx_v, urows_v, irows_v, out_v, sem_u, sem_i):
    cid = lax.axis_index("c")
    sid = lax.axis_index("s")
    wid = sid * NC + cid
    base = wid * BPW

    for j in range(NCHUNK):
        pltpu.sync_copy(upos_hbm.at[pl.ds(base + j * CHUNK, CHUNK)],
                        uidx_v.at[j])
        pltpu.sync_copy(ipos_hbm.at[pl.ds(base + j * CHUNK, CHUNK)],
                        iidx_v.at[j])
    copies = []
    for j in range(NCHUNK):
        copies.append(pltpu.async_copy(
            ue_hbm.at[uidx_v.at[j]], urows_v.at[pl.ds(j * CHUNK, CHUNK)],
            sem_u))
        copies.append(pltpu.async_copy(
            ie_hbm.at[iidx_v.at[j]], irows_v.at[pl.ds(j * CHUNK, CHUNK)],
            sem_i))
    for c in copies:
        c.wait()

    iota = lax.iota(jnp.int32, L)

    def body(c, carry):
        rows = c * L + iota
        acc = jnp.zeros((L,), jnp.float32)
        for k in range(K):
            col = jnp.full((L,), k, jnp.int32)
            uk = plsc.load_gather(urows_v, [rows, col])
            ik = plsc.load_gather(irows_v, [rows, col])
            acc = acc + uk * ik
        out_v[pl.ds(c * L, L)] = 1.0 / (1.0 + jnp.exp(-acc))
        return carry

    lax.fori_loop(0, BPW // L, body, 0)
    pltpu.sync_copy(out_v, out_hbm.at[pl.ds(base, BPW)])


def _run_flags(sorted_idx):
    bkt = sorted_idx >> 7
    head = jnp.concatenate([
        jnp.ones((1,), jnp.int32),
        (bkt[1:] != bkt[:-1]).astype(jnp.int32)])
    pos = lax.iota(jnp.int32, B)
    head = jnp.where(lax.rem(pos, BPW) == 0, 1, head)
    wd = jnp.cumsum(head) - 1
    return head, wd.astype(jnp.int32)


def kernel(user, item, user_table, item_table):
    user = user.astype(jnp.int32)
    item = item.astype(jnp.int32)
    pu = jnp.argsort(user)
    pi = jnp.argsort(item)
    su = user[pu]
    si = item[pi]
    nfu, wdu = _run_flags(su)
    nfi, wdi = _run_flags(si)
    upos = jnp.argsort(pu).astype(jnp.int32)
    ipos = jnp.argsort(pi).astype(jnp.int32)

    mesh = plsc.VectorSubcoreMesh(core_axis_name="c", subcore_axis_name="s")
    extract = pl.kernel(
        _extract_body,
        mesh=mesh,
        out_type=[jax.ShapeDtypeStruct((B * K,), jnp.float32),
                  jax.ShapeDtypeStruct((B * K,), jnp.float32)],
        scratch_types=[
            pltpu.VMEM((BPW,), jnp.int32),
            pltpu.VMEM((BPW,), jnp.int32),
            pltpu.VMEM((BPW,), jnp.int32),
            pltpu.VMEM((NSLOT, K, W), jnp.float32),
            pltpu.VMEM((BPW * K,), jnp.float32),
            pltpu.SemaphoreType.DMA,
        ],
        compiler_params=pltpu.CompilerParams(needs_layout_passes=False),
    )
    uemb, iemb = extract(su, si, nfu, nfi, wdu, wdi,
                         user_table.T, item_table.T)

    pair = pl.kernel(
        _pair_body,
        mesh=mesh,
        out_type=jax.ShapeDtypeStruct((B,), jnp.float32),
        scratch_types=[
            pltpu.VMEM((NCHUNK, CHUNK), jnp.int32),
            pltpu.VMEM((NCHUNK, CHUNK), jnp.int32),
            pltpu.VMEM((BPW, K), jnp.float32),
            pltpu.VMEM((BPW, K), jnp.float32),
            pltpu.VMEM((BPW,), jnp.float32),
            pltpu.SemaphoreType.DMA,
            pltpu.SemaphoreType.DMA,
        ],
        compiler_params=pltpu.CompilerParams(
            needs_layout_passes=False, use_tc_tiling_on_sc=False),
    )
    return pair(upos, ipos, uemb.reshape(B, K), iemb.reshape(B, K))


# speedup vs baseline: 1.1674x; 1.1674x over previous
import jax
import jax.numpy as jnp
from jax import lax
from jax.experimental import pallas as pl
from jax.experimental.pallas import tpu as pltpu
from jax.experimental.pallas import tpu_sc as plsc

NC = 2
NS = 16
L = 16
NW = NC * NS
B = 16384
K = 32
BPW = B // NW
W = 128
NSLOT = 17
CHUNK = 128
NCHUNK = BPW // CHUNK


def _extract_body(su_hbm, si_hbm, nfu_hbm, nfi_hbm, wdu_hbm, wdi_hbm,
                  ut_hbm, it_hbm, uemb_hbm, iemb_hbm,
                  idx_v, nf_v, wd_v, slabs, ebuf, sem):
    cid = lax.axis_index("c")
    sid = lax.axis_index("s")
    wid = sid * NC + cid
    base = wid * BPW
    iota = lax.iota(jnp.int32, L)

    def one_pass(sv_hbm, nf_hbm, wdx_hbm, tab, out_hbm):
        pltpu.sync_copy(sv_hbm.at[pl.ds(base, BPW)], idx_v)
        pltpu.sync_copy(nf_hbm.at[pl.ds(base, BPW)], nf_v)
        pltpu.sync_copy(wdx_hbm.at[pl.ds(base, BPW)], wd_v)

        H = L // 2

        def load16(g):
            return (idx_v[pl.ds(g * L, L)], nf_v[pl.ds(g * L, L)],
                    wd_v[pl.ds(g * L, L)])

        def fire(svv, nfv, wdv, lo_lane):
            for i in range(lo_lane, lo_lane + H):
                r = svv[i]
                slot = lax.rem(wdv[i], NSLOT)
                cb = pl.multiple_of(r - lax.rem(r, W), W)

                @pl.when(nfv[i] == 1)
                def _fire():
                    pltpu.async_copy(tab.at[:, pl.ds(cb, W)],
                                     slabs.at[slot], sem)

        def drain(nfv, lo_lane):
            for i in range(lo_lane, lo_lane + H):
                @pl.when(nfv[i] == 1)
                def _drain():
                    pltpu.make_async_copy(tab.at[:, pl.ds(0, W)],
                                          slabs.at[0], sem).wait()

        def extract(svv, wdv, lo_lane, g):
            for i in range(lo_lane, lo_lane + H):
                r = svv[i]
                slot = lax.rem(wdv[i], NSLOT)
                j = lax.rem(r, W)
                sl = jnp.full((L,), slot, jnp.int32)
                jj = jnp.full((L,), j, jnp.int32)
                lo = plsc.load_gather(slabs, [sl, iota, jj])
                hi = plsc.load_gather(slabs, [sl, iota + L, jj])
                q = g * L + i
                ebuf[pl.ds(q * K, L)] = lo
                ebuf[pl.ds(q * K + L, L)] = hi

        nblk = BPW // L
        v0 = load16(0)
        fire(v0[0], v0[1], v0[2], 0)

        def block(g, carry):
            svv, nfv, wdv = load16(g)
            fire(svv, nfv, wdv, H)
            drain(nfv, 0)
            extract(svv, wdv, 0, g)
            gn = lax.min(g + 1, nblk - 1)
            svn, nfn, wdn = load16(gn)

            @pl.when(g + 1 < nblk)
            def _prefire():
                fire(svn, nfn, wdn, 0)

            drain(nfv, H)
            extract(svv, wdv, H, g)
            return carry

        lax.fori_loop(0, nblk, block, 0)
        pltpu.sync_copy(ebuf, out_hbm.at[pl.ds(base * K, BPW * K)])

    one_pass(su_hbm, nfu_hbm, wdu_hbm, ut_hbm, uemb_hbm)
    one_pass(si_hbm, nfi_hbm, wdi_hbm, it_hbm, iemb_hbm)


def _pair_body(upos_hbm, ipos_hbm, ue_hbm, ie_hbm, out_hbm,
               uidx_v, iidx_v, urows_v, irows_v, out_v, sem_u, sem_i):
    cid = lax.axis_index("c")
    sid = lax.axis_index("s")
    wid = sid * NC + cid
    base = wid * BPW

    for j in range(NCHUNK):
        pltpu.sync_copy(upos_hbm.at[pl.ds(base + j * CHUNK, CHUNK)],
                        uidx_v.at[j])
        pltpu.sync_copy(ipos_hbm.at[pl.ds(base + j * CHUNK, CHUNK)],
                        iidx_v.at[j])
    copies = []
    for j in range(NCHUNK):
        copies.append(pltpu.async_copy(
            ue_hbm.at[uidx_v.at[j]], urows_v.at[pl.ds(j * CHUNK, CHUNK)],
            sem_u))
        copies.append(pltpu.async_copy(
            ie_hbm.at[iidx_v.at[j]], irows_v.at[pl.ds(j * CHUNK, CHUNK)],
            sem_i))
    for c in copies:
        c.wait()

    iota = lax.iota(jnp.int32, L)

    def body(c, carry):
        rows = c * L + iota
        acc = jnp.zeros((L,), jnp.float32)
        for k in range(K):
            col = jnp.full((L,), k, jnp.int32)
            uk = plsc.load_gather(urows_v, [rows, col])
            ik = plsc.load_gather(irows_v, [rows, col])
            acc = acc + uk * ik
        out_v[pl.ds(c * L, L)] = 1.0 / (1.0 + jnp.exp(-acc))
        return carry

    lax.fori_loop(0, BPW // L, body, 0)
    pltpu.sync_copy(out_v, out_hbm.at[pl.ds(base, BPW)])


def _run_flags(sorted_idx):
    bkt = sorted_idx >> 7
    head = jnp.concatenate([
        jnp.ones((1,), jnp.int32),
        (bkt[1:] != bkt[:-1]).astype(jnp.int32)])
    pos = lax.iota(jnp.int32, B)
    head = jnp.where(lax.rem(pos, BPW) == 0, 1, head)
    wd = jnp.cumsum(head) - 1
    return head, wd.astype(jnp.int32)


def kernel(user, item, user_table, item_table):
    user = user.astype(jnp.int32)
    item = item.astype(jnp.int32)
    pu = jnp.argsort(user)
    pi = jnp.argsort(item)
    su = user[pu]
    si = item[pi]
    nfu, wdu = _run_flags(su)
    nfi, wdi = _run_flags(si)
    upos = jnp.argsort(pu).astype(jnp.int32)
    ipos = jnp.argsort(pi).astype(jnp.int32)

    mesh = plsc.VectorSubcoreMesh(core_axis_name="c", subcore_axis_name="s")
    extract = pl.kernel(
        _extract_body,
        mesh=mesh,
        out_type=[jax.ShapeDtypeStruct((B * K,), jnp.float32),
                  jax.ShapeDtypeStruct((B * K,), jnp.float32)],
        scratch_types=[
            pltpu.VMEM((BPW,), jnp.int32),
            pltpu.VMEM((BPW,), jnp.int32),
            pltpu.VMEM((BPW,), jnp.int32),
            pltpu.VMEM((NSLOT, K, W), jnp.float32),
            pltpu.VMEM((BPW * K,), jnp.float32),
            pltpu.SemaphoreType.DMA,
        ],
        compiler_params=pltpu.CompilerParams(needs_layout_passes=False),
    )
    uemb, iemb = extract(su, si, nfu, nfi, wdu, wdi,
                         user_table.T, item_table.T)

    pair = pl.kernel(
        _pair_body,
        mesh=mesh,
        out_type=jax.ShapeDtypeStruct((B,), jnp.float32),
        scratch_types=[
            pltpu.VMEM((NCHUNK, CHUNK), jnp.int32),
            pltpu.VMEM((NCHUNK, CHUNK), jnp.int32),
            pltpu.VMEM((BPW, K), jnp.float32),
            pltpu.VMEM((BPW, K), jnp.float32),
            pltpu.VMEM((BPW,), jnp.float32),
            pltpu.SemaphoreType.DMA,
            pltpu.SemaphoreType.DMA,
        ],
        compiler_params=pltpu.CompilerParams(
            needs_layout_passes=False, use_tc_tiling_on_sc=False),
    )
    return pair(upos, ipos, uemb.reshape(B, K), iemb.reshape(B, K))
